# Initial kernel scaffold; baseline (speedup 1.0000x reference)
import jax
import jax.numpy as jnp
from jax.experimental import pallas as pl


def kernel(coords, feats):
    raise NotImplementedError("write your pallas kernel here")


# speedup vs baseline: 2.5249x; 2.5249x over previous
import jax
import jax.numpy as jnp
from jax import lax
from jax.experimental import pallas as pl
from jax.experimental.pallas import tpu as pltpu
from jax.experimental.pallas import tpu_sc as plsc

_RES = 64
_RV = _RES + 1
_NV = _RV ** 3
_M = 10
_N = 262144
_SDF_BIAS = -1.0 / _RES

_NC, _NS = 2, 16

_Q = 68736
_DUMPQ = _Q
_HACCQ = 69632
_ZCHQ = _HACCQ // _NS

_NVP = 4 * _Q

_CPT = _N // _NS
_CW = 256
_NSW = _CPT // _CW
_NPAIR = _CW // 32

_ZW = 128
_NWQ = 537

_OFF = tuple((c & 1) * _RV * _RV + ((c >> 1) & 1) * _RV + (c >> 2)
             for c in range(8))


def _sc_body(xs_hbm, ys_hbm, zs_hbm, feats_hbm, zeros_hbm,
             sums_hbm, cnt_hbm,
             featbuf0, featbuf1, xsb, ysb, zsb, idxbuf, onesb, zrow,
             sem, acc, cnt):
    cid = lax.axis_index("c")
    sid = lax.axis_index("s")
    iota = lax.iota(jnp.int32, 16)
    zerov = jnp.zeros((16,), jnp.float32)

    pats = [jnp.where(iota < 8, 2 * p, 2 * p + 1) for p in range(8)]
    c8 = iota % 8
    offpat = ((c8 & 1) * (_RV * _RV) + ((c8 >> 1) & 1) * _RV + (c8 >> 2))
    _dnums = lax.GatherDimensionNumbers(
        offset_dims=(), collapsed_slice_dims=(0,), start_index_map=(0,))

    def _vperm(x, pat):
        return lax.gather(x, pat[:, None], dimension_numbers=_dnums,
                          slice_sizes=(1,),
                          mode=lax.GatherScatterMode.PROMISE_IN_BOUNDS)

    for i in range(8):
        onesb[pl.ds(i * 16, 16)] = jnp.ones((16,), jnp.float32)

    cube0 = sid * _CPT
    zbase = sid * _ZCHQ
    nwin = jnp.where(sid < 9, 34, 33)
    wout0 = (sid * 33 + jnp.minimum(sid, 9)) * _ZW

    for t in range(2):
        qi = cid * 2 + t
        qbase = qi * _Q
        hloc_u = lax.convert_element_type(
            jnp.minimum(_NV - qbase, _Q), jnp.uint32)

        def _zrow_fill(i, carry):
            zrow[pl.ds(i * 16, 16)] = zerov
            return carry
        lax.fori_loop(0, _ZW // 16, _zrow_fill, 0)
        pltpu.sync_copy(zeros_hbm, featbuf1)

        def _zacc(k, carry):
            pltpu.sync_copy(featbuf1, acc.at[pl.ds(zbase + k * _ZW, _ZW)])
            pltpu.sync_copy(zrow, cnt.at[pl.ds(zbase + k * _ZW, _ZW)])
            return carry
        lax.fori_loop(0, _ZCHQ // _ZW, _zacc, 0)

        plsc.subcore_barrier()

        def _idx_window(q, par, salt):
            x = xsb[pl.ds(q, 16)]
            y = ysb[pl.ds(q, 16)]
            z = zsb[pl.ds(q, 16)]
            base = (x * (_RV * _RV) + y * _RV + z) - qbase
            for p in range(8):
                bg = _vperm(base, pats[p])
                sel = bg + offpat
                valid = plsc.bitcast(sel, jnp.uint32) < hloc_u
                dvec = (_DUMPQ + ((salt * 8 + p) % 16) * 16) + iota
                sel = jnp.where(valid, sel, dvec)
                idxbuf.at[par][pl.ds(p * 16, 16)] = sel

        def _sw(sw, carry):
            cb0 = cube0 + sw * _CW
            pltpu.sync_copy(xs_hbm.at[pl.ds(cb0, _CW)], xsb)
            pltpu.sync_copy(ys_hbm.at[pl.ds(cb0, _CW)], ysb)
            pltpu.sync_copy(zs_hbm.at[pl.ds(cb0, _CW)], zsb)

            def _pair(jp, c2):
                q0 = jp * 32
                r0 = (cb0 + q0) * 8
                in0 = pltpu.async_copy(feats_hbm.at[pl.ds(r0, 128)],
                                       featbuf0, sem)
                in1 = pltpu.async_copy(feats_hbm.at[pl.ds(r0 + 128, 128)],
                                       featbuf1, sem)
                _idx_window(q0, 0, jp)
                in0.wait()
                s0a = pltpu.async_copy(featbuf0, acc.at[idxbuf.at[0]],
                                       sem, add=True)
                s0b = pltpu.async_copy(onesb, cnt.at[idxbuf.at[0]],
                                       sem, add=True)
                _idx_window(q0 + 16, 1, jp + 1)
                in1.wait()
                s1a = pltpu.async_copy(featbuf1, acc.at[idxbuf.at[1]],
                                       sem, add=True)
                s1b = pltpu.async_copy(onesb, cnt.at[idxbuf.at[1]],
                                       sem, add=True)
                s0a.wait()
                s0b.wait()
                s1a.wait()
                s1b.wait()
                return c2
            lax.fori_loop(0, _NPAIR, _pair, 0)
            return carry
        lax.fori_loop(0, _NSW, _sw, 0)

        plsc.subcore_barrier()

        def _outwin(k, carry):
            @pl.when(k < nwin)
            def _():
                s = wout0 + k * _ZW
                pltpu.sync_copy(acc.at[pl.ds(s, _ZW)], featbuf0)
                pltpu.sync_copy(cnt.at[pl.ds(s, _ZW)], zrow)
                pltpu.sync_copy(featbuf0,
                                sums_hbm.at[pl.ds(qbase + s, _ZW)])
                pltpu.sync_copy(zrow, cnt_hbm.at[pl.ds(qbase + s, _ZW)])
            return carry
        lax.fori_loop(0, 34, _outwin, 0)

        plsc.subcore_barrier()


_DBLK = 2048


def _div_body(s_ref, c_ref, o_ref):
    s = s_ref[...]
    c = c_ref[...]
    cm = jnp.maximum(c, 1.0)
    col = lax.broadcasted_iota(jnp.int32, (_DBLK, _M), 1)
    bias = jnp.where((col == 0) & (c > 0.0), jnp.float32(_SDF_BIAS),
                     jnp.float32(0.0))
    o_ref[...] = s / cm + bias


@jax.jit
def _impl(coords, feats):
    xs = coords[:, 0]
    ys = coords[:, 1]
    zs = coords[:, 2]
    feats2d = feats.reshape(_N * 8, _M)
    zeros = jnp.zeros((_ZW, _M), jnp.float32)
    mesh = plsc.VectorSubcoreMesh(core_axis_name="c", subcore_axis_name="s",
                                  num_cores=_NC, num_subcores=_NS)
    scatter = pl.kernel(
        _sc_body,
        out_type=(jax.ShapeDtypeStruct((_NVP, _M), jnp.float32),
                  jax.ShapeDtypeStruct((_NVP,), jnp.float32)),
        mesh=mesh,
        compiler_params=pltpu.CompilerParams(needs_layout_passes=False,
                                             use_tc_tiling_on_sc=False),
        scratch_types=[
            pltpu.VMEM((_ZW, _M), jnp.float32),
            pltpu.VMEM((_ZW, _M), jnp.float32),
            pltpu.VMEM((_CW,), jnp.int32),
            pltpu.VMEM((_CW,), jnp.int32),
            pltpu.VMEM((_CW,), jnp.int32),
            pltpu.VMEM((2, 128), jnp.int32),
            pltpu.VMEM((128,), jnp.float32),
            pltpu.VMEM((_ZW,), jnp.float32),
            pltpu.SemaphoreType.DMA,
            pltpu.VMEM_SHARED((_HACCQ, _M), jnp.float32),
            pltpu.VMEM_SHARED((_HACCQ,), jnp.float32),
        ],
    )
    sums, counts = scatter(xs, ys, zs, feats2d, zeros)
    counts2d = counts.reshape(_NVP, 1)
    dense = pl.pallas_call(
        _div_body,
        grid=((_NVP + _DBLK - 1) // _DBLK,),
        in_specs=[pl.BlockSpec((_DBLK, _M), lambda i: (i, 0)),
                  pl.BlockSpec((_DBLK, 1), lambda i: (i, 0))],
        out_specs=pl.BlockSpec((_DBLK, _M), lambda i: (i, 0)),
        out_shape=jax.ShapeDtypeStruct((_NVP, _M), jnp.float32),
    )(sums, counts2d)
    return dense[:_NV]


def kernel(coords, feats):
    return _impl(coords, feats)
